# Initial kernel scaffold; baseline (speedup 1.0000x reference)
import jax
import jax.numpy as jnp
from jax.experimental import pallas as pl


def kernel(x, table, W, b):
    raise NotImplementedError("write your pallas kernel here")


# speedup vs baseline: 8.9272x; 8.9272x over previous
import functools

import jax
import jax.numpy as jnp
from jax import lax
from jax.experimental import pallas as pl
from jax.experimental.pallas import tpu as pltpu
from jax.experimental.pallas import tpu_sc as plsc

_NC = 2
_NS = 16
_NW = _NC * _NS
_CB = 8


def _sc_pool(x2, table):
    B2, H2 = x2.shape
    B = B2 // 2
    H = 2 * H2
    V, D = table.shape
    per_w = B // _NW
    n_chunks = per_w // _CB
    n_pairs = n_chunks // 2
    inv_h = jnp.float32(1.0 / H)

    mesh = plsc.VectorSubcoreMesh(core_axis_name="c", subcore_axis_name="s")

    @functools.partial(
        pl.kernel,
        mesh=mesh,
        compiler_params=pltpu.CompilerParams(use_tc_tiling_on_sc=False),
        out_type=jax.ShapeDtypeStruct((B, D), jnp.float32),
        scratch_types=[
            pltpu.VMEM((2, 2 * _CB, H2), jnp.int32),
            pltpu.VMEM((2, _CB, H, D), jnp.float32),
            pltpu.VMEM((_CB, D), jnp.float32),
            pltpu.SemaphoreType.DMA,
            pltpu.SemaphoreType.DMA,
        ],
    )
    def pool_kernel(x2_hbm, table_hbm, out_hbm, idx_v, rows_v, pooled_v, sem0, sem1):
        wid = lax.axis_index("s") * _NC + lax.axis_index("c")
        base = wid * per_w
        sems = (sem0, sem1)

        def load_and_fire(t, buf):
            r0 = (base + t * _CB) * 2
            pltpu.sync_copy(x2_hbm.at[pl.ds(r0, 2 * _CB)], idx_v.at[buf])
            for e in range(_CB):
                pltpu.async_copy(
                    table_hbm.at[idx_v.at[buf, 2 * e]],
                    rows_v.at[buf, e, pl.ds(0, H2)],
                    sems[buf],
                )
                pltpu.async_copy(
                    table_hbm.at[idx_v.at[buf, 2 * e + 1]],
                    rows_v.at[buf, e, pl.ds(H2, H2)],
                    sems[buf],
                )

        def wait_gathers(buf):
            for e in range(_CB):
                pltpu.make_async_copy(
                    table_hbm.at[idx_v.at[buf, 2 * e]],
                    rows_v.at[buf, e, pl.ds(0, H2)],
                    sems[buf],
                ).wait()
                pltpu.make_async_copy(
                    table_hbm.at[idx_v.at[buf, 2 * e + 1]],
                    rows_v.at[buf, e, pl.ds(H2, H2)],
                    sems[buf],
                ).wait()

        def reduce_chunk(t, buf):
            for e in range(_CB):
                zero = jnp.zeros((D,), jnp.float32)

                def body(j, accs, _e=e, _buf=buf):
                    a0, a1, a2, a3 = accs
                    j4 = 4 * j
                    a0 = a0 + rows_v[_buf, _e, j4]
                    a1 = a1 + rows_v[_buf, _e, j4 + 1]
                    a2 = a2 + rows_v[_buf, _e, j4 + 2]
                    a3 = a3 + rows_v[_buf, _e, j4 + 3]
                    return (a0, a1, a2, a3)

                a0, a1, a2, a3 = lax.fori_loop(0, H // 4, body, (zero,) * 4)
                pooled_v[e] = ((a0 + a1) + (a2 + a3)) * inv_h
            pltpu.sync_copy(pooled_v, out_hbm.at[pl.ds(base + t * _CB, _CB)])

        load_and_fire(0, 0)

        def pair_body(p, carry):
            t0 = 2 * p
            load_and_fire(t0 + 1, 1)
            wait_gathers(0)
            reduce_chunk(t0, 0)

            @pl.when(p < n_pairs - 1)
            def _():
                load_and_fire(t0 + 2, 0)

            wait_gathers(1)
            reduce_chunk(t0 + 1, 1)
            return carry

        lax.fori_loop(0, n_pairs, pair_body, 0)

    return pool_kernel(x2, table)


def _tc_head(pooled, W, b2):
    B, D = pooled.shape
    C = W.shape[1]
    BT = 2048

    def head_body(p_ref, w_ref, b_ref, o_ref):
        logits = (
            jnp.dot(p_ref[...], w_ref[...], preferred_element_type=jnp.float32)
            + b_ref[...]
        )
        m = jnp.max(logits, axis=-1, keepdims=True)
        e = jnp.exp(logits - m)
        o_ref[...] = e / jnp.sum(e, axis=-1, keepdims=True)

    return pl.pallas_call(
        head_body,
        grid=(B // BT,),
        in_specs=[
            pl.BlockSpec((BT, D), lambda i: (i, 0)),
            pl.BlockSpec((D, C), lambda i: (0, 0)),
            pl.BlockSpec((1, C), lambda i: (0, 0)),
        ],
        out_specs=pl.BlockSpec((BT, C), lambda i: (i, 0)),
        out_shape=jax.ShapeDtypeStruct((B, C), jnp.float32),
    )(pooled, W, b2)


def kernel(x, table, W, b):
    B, H = x.shape
    pooled = _sc_pool(x.reshape(2 * B, H // 2), table)
    return _tc_head(pooled, W, b.reshape(1, -1))
